# baseline (speedup 1.0000x reference)
import functools

import jax
import jax.numpy as jnp
from jax import lax
from jax.experimental import pallas as pl
from jax.experimental.pallas import tpu as pltpu
from jax.experimental.pallas import tpu_sc as plsc

VOCAB = 1000000
EMBED_DIM = 64
BATCH = 4096
SEQ = 200
BASE = 10000.0
HALF = EMBED_DIM // 2

NC = 2
NS = 16
NW = NC * NS
ROWS_PER_W = BATCH // NW
IDXROWS_PER_W = 2 * ROWS_PER_W

_mesh = plsc.VectorSubcoreMesh(core_axis_name="c", subcore_axis_name="s")


@functools.partial(
    pl.kernel,
    mesh=_mesh,
    out_type=jax.ShapeDtypeStruct((BATCH, SEQ, EMBED_DIM), jnp.float32),
    scratch_types=[
        pltpu.VMEM((IDXROWS_PER_W, SEQ // 2), jnp.int32),
        pltpu.VMEM((SEQ, EMBED_DIM), jnp.float32),
        pltpu.VMEM((SEQ, EMBED_DIM), jnp.float32),
        pltpu.VMEM((SEQ, EMBED_DIM), jnp.float32),
        pltpu.SemaphoreType.DMA,
        pltpu.SemaphoreType.DMA,
    ],
    compiler_params=pltpu.CompilerParams(use_tc_tiling_on_sc=False),
)
def _rope_embed(x_hbm, table_hbm, rope_hbm, out_hbm,
                idx_v, buf0, buf1, rope_v, sem0, sem1):
    wid = lax.axis_index("s") * NC + lax.axis_index("c")
    base = wid * ROWS_PER_W

    pltpu.sync_copy(x_hbm.at[pl.ds(base * 2, IDXROWS_PER_W)], idx_v)
    pltpu.sync_copy(rope_hbm, rope_v)

    bufs = (buf0, buf1)
    sems = (sem0, sem1)

    def start_gather(c, buf, sem):
        pltpu.async_copy(table_hbm.at[idx_v.at[2 * c]],
                         buf.at[pl.ds(0, SEQ // 2)], sem)
        pltpu.async_copy(table_hbm.at[idx_v.at[2 * c + 1]],
                         buf.at[pl.ds(SEQ // 2, SEQ // 2)], sem)

    def wait_gather(buf, sem):
        pltpu.make_async_copy(table_hbm.at[pl.ds(0, SEQ)], buf, sem).wait()

    def rope_chunk(buf):
        def row(r, carry):
            ev0 = buf[r, pl.ds(0, 16)]
            ev1 = buf[r, pl.ds(16, 16)]
            od0 = buf[r, pl.ds(32, 16)]
            od1 = buf[r, pl.ds(48, 16)]
            c0 = rope_v[r, pl.ds(0, 16)]
            c1 = rope_v[r, pl.ds(16, 16)]
            s0 = rope_v[r, pl.ds(32, 16)]
            s1 = rope_v[r, pl.ds(48, 16)]
            buf[r, pl.ds(0, 16)] = ev0 * c0 - od0 * s0
            buf[r, pl.ds(16, 16)] = ev1 * c1 - od1 * s1
            buf[r, pl.ds(32, 16)] = ev0 * s0 + od0 * c0
            buf[r, pl.ds(48, 16)] = ev1 * s1 + od1 * c1
            return carry
        lax.fori_loop(0, SEQ, row, 0)

    start_gather(0, buf0, sem0)

    def outer(g, carry):
        for b in range(2):
            c = 2 * g + b
            wait_gather(bufs[b], sems[b])

            @pl.when(c + 1 < ROWS_PER_W)
            def _():
                start_gather(c + 1, bufs[1 - b], sems[1 - b])

            rope_chunk(bufs[b])
            pltpu.sync_copy(bufs[b], out_hbm.at[base + c])
        return carry

    lax.fori_loop(0, ROWS_PER_W // 2, outer, 0)


def _rope_table():
    positions = jnp.arange(SEQ, dtype=jnp.float32)[:, None]
    freqs_indices = jnp.arange(HALF, dtype=jnp.float32)
    freqs = 1.0 / (BASE ** (freqs_indices / EMBED_DIM))
    angles = positions * freqs
    return jnp.concatenate([jnp.cos(angles), jnp.sin(angles)], axis=-1)


@jax.jit
def kernel(x, table):
    x2 = x.astype(jnp.int32).reshape(BATCH * 2, SEQ // 2)
    return _rope_embed(x2, table, _rope_table())


# speedup vs baseline: 1.0411x; 1.0411x over previous
import functools

import jax
import jax.numpy as jnp
from jax import lax
from jax.experimental import pallas as pl
from jax.experimental.pallas import tpu as pltpu
from jax.experimental.pallas import tpu_sc as plsc

VOCAB = 1000000
EMBED_DIM = 64
BATCH = 4096
SEQ = 200
BASE = 10000.0
HALF = EMBED_DIM // 2

NC = 2
NS = 16
NW = NC * NS
ROWS_PER_W = BATCH // NW
IDXROWS_PER_W = 2 * ROWS_PER_W

_mesh = plsc.VectorSubcoreMesh(core_axis_name="c", subcore_axis_name="s")


@functools.partial(
    pl.kernel,
    mesh=_mesh,
    out_type=jax.ShapeDtypeStruct((BATCH, SEQ, EMBED_DIM), jnp.float32),
    scratch_types=[
        pltpu.VMEM((IDXROWS_PER_W, SEQ // 2), jnp.int32),
        pltpu.VMEM((SEQ, EMBED_DIM), jnp.float32),
        pltpu.VMEM((SEQ, EMBED_DIM), jnp.float32),
        pltpu.VMEM((SEQ, EMBED_DIM), jnp.float32),
        pltpu.SemaphoreType.DMA,
        pltpu.SemaphoreType.DMA,
    ],
    compiler_params=pltpu.CompilerParams(use_tc_tiling_on_sc=False),
)
def _rope_embed(x_hbm, table_hbm, rope_hbm, out_hbm,
                idx_v, buf0, buf1, rope_v, sem0, sem1):
    wid = lax.axis_index("s") * NC + lax.axis_index("c")
    base = wid * ROWS_PER_W

    pltpu.sync_copy(x_hbm.at[pl.ds(base * 2, IDXROWS_PER_W)], idx_v)
    pltpu.sync_copy(rope_hbm, rope_v)

    bufs = (buf0, buf1)
    sems = (sem0, sem1)

    def start_gather(c, buf, sem):
        pltpu.async_copy(table_hbm.at[idx_v.at[2 * c]],
                         buf.at[pl.ds(0, SEQ // 2)], sem)
        pltpu.async_copy(table_hbm.at[idx_v.at[2 * c + 1]],
                         buf.at[pl.ds(SEQ // 2, SEQ // 2)], sem)

    def wait_gather(buf, sem):
        pltpu.make_async_copy(table_hbm.at[pl.ds(0, SEQ)], buf, sem).wait()

    def rope_chunk(buf):
        @plsc.parallel_loop(0, SEQ, unroll=8)
        def _row(r):
            ev0 = buf[r, pl.ds(0, 16)]
            ev1 = buf[r, pl.ds(16, 16)]
            od0 = buf[r, pl.ds(32, 16)]
            od1 = buf[r, pl.ds(48, 16)]
            c0 = rope_v[r, pl.ds(0, 16)]
            c1 = rope_v[r, pl.ds(16, 16)]
            s0 = rope_v[r, pl.ds(32, 16)]
            s1 = rope_v[r, pl.ds(48, 16)]
            buf[r, pl.ds(0, 16)] = ev0 * c0 - od0 * s0
            buf[r, pl.ds(16, 16)] = ev1 * c1 - od1 * s1
            buf[r, pl.ds(32, 16)] = ev0 * s0 + od0 * c0
            buf[r, pl.ds(48, 16)] = ev1 * s1 + od1 * c1

    start_gather(0, buf0, sem0)

    def outer(g, carry):
        for b in range(2):
            c = 2 * g + b
            wait_gather(bufs[b], sems[b])

            @pl.when(c + 1 < ROWS_PER_W)
            def _():
                start_gather(c + 1, bufs[1 - b], sems[1 - b])

            rope_chunk(bufs[b])
            pltpu.sync_copy(bufs[b], out_hbm.at[base + c])
        return carry

    lax.fori_loop(0, ROWS_PER_W // 2, outer, 0)


def _rope_table():
    positions = jnp.arange(SEQ, dtype=jnp.float32)[:, None]
    freqs_indices = jnp.arange(HALF, dtype=jnp.float32)
    freqs = 1.0 / (BASE ** (freqs_indices / EMBED_DIM))
    angles = positions * freqs
    return jnp.concatenate([jnp.cos(angles), jnp.sin(angles)], axis=-1)


@jax.jit
def kernel(x, table):
    x2 = x.astype(jnp.int32).reshape(BATCH * 2, SEQ // 2)
    return _rope_embed(x2, table, _rope_table())
